# baseline (speedup 1.0000x reference)
import functools

import jax
import jax.numpy as jnp
from jax import lax
from jax.experimental import pallas as pl
from jax.experimental.pallas import tpu as pltpu
from jax.experimental.pallas import tpu_sc as plsc


_LANES = 16
_T_SC = 768

_GATHER_DNUMS = lax.GatherDimensionNumbers(
    offset_dims=(), collapsed_slice_dims=(0,), start_index_map=(0,)
)


def _vgather(v, idx):
    return lax.gather(
        v,
        idx[:, None],
        _GATHER_DNUMS,
        slice_sizes=(1,),
        mode=lax.GatherScatterMode.PROMISE_IN_BOUNDS,
    )


def _argmax_body(x_ref, out_ref):
    x = x_ref[...]
    m = jnp.max(x, axis=-1, keepdims=True)
    ii = lax.broadcasted_iota(jnp.int32, x.shape, 2)
    C = x.shape[-1]
    idx = jnp.min(jnp.where(x >= m, ii, C), axis=-1).astype(jnp.int32)
    out_ref[...] = idx.T


def _sc_argmax_body(T1, N, C, em_hbm, out_hbm, buf0, buf1, res_v, sem0, sem1):
    wid = lax.axis_index("s") * 2 + lax.axis_index("c")
    tpw = _T_SC // 32
    t0 = T1 + wid * tpw
    lane = lax.iota(jnp.int32, _LANES)
    nchunks = C // _LANES
    neg_inf = jnp.float32(-jnp.inf)

    pltpu.async_copy(em_hbm.at[t0], buf0, sem0)
    pltpu.async_copy(em_hbm.at[t0 + 1], buf1, sem1)

    unroll = 8
    idx_const = [lane + s * _LANES for s in range(unroll)]

    def _comb(a, b):
        va, ia = a
        vb, ib = b
        gt = vb > va
        return jnp.where(gt, vb, va), jnp.where(gt, ib, ia)

    def frame_argmax(buf, n):
        def chunk8(c8, carry):
            m, ai = carry
            cbase = c8 * (unroll * _LANES)
            leaves = [
                (buf[n, pl.ds(cbase + s * _LANES, _LANES)], idx_const[s])
                for s in range(unroll)
            ]
            l4 = [_comb(leaves[2 * i], leaves[2 * i + 1]) for i in range(4)]
            l2 = [_comb(l4[0], l4[1]), _comb(l4[2], l4[3])]
            v8, i8 = _comb(l2[0], l2[1])
            gt = v8 > m
            m = jnp.where(gt, v8, m)
            ai = jnp.where(gt, i8 + cbase, ai)
            return m, ai

        m0 = jnp.zeros((_LANES,), jnp.float32) + neg_inf
        m, ai = lax.fori_loop(
            0, nchunks // unroll, chunk8, (m0, jnp.zeros((_LANES,), jnp.int32))
        )
        mx = jnp.max(m)
        return jnp.min(jnp.where(m == mx, ai, C))

    def process(buf, j):
        def per_frame(n, carry):
            lo, hi = carry
            res = frame_argmax(buf, n)
            sel = lane == (n % _LANES)
            lo = jnp.where(sel & (n < _LANES), res, lo)
            hi = jnp.where(sel & (n >= _LANES), res, hi)
            return lo, hi

        z = jnp.zeros((_LANES,), jnp.int32)
        lo, hi = lax.fori_loop(0, N, per_frame, (z, z))
        res_v[pl.ds(j * N, _LANES)] = lo
        res_v[pl.ds(j * N + _LANES, _LANES)] = hi

    def outer(k, _):
        j0 = 2 * k
        t = t0 + j0
        pltpu.make_async_copy(em_hbm.at[t], buf0, sem0).wait()
        process(buf0, j0)

        @pl.when(j0 + 2 < tpw)
        def _():
            pltpu.async_copy(em_hbm.at[t + 2], buf0, sem0)

        pltpu.make_async_copy(em_hbm.at[t + 1], buf1, sem1).wait()
        process(buf1, j0 + 1)

        @pl.when(j0 + 3 < tpw)
        def _():
            pltpu.async_copy(em_hbm.at[t + 3], buf1, sem1)

        return 0

    lax.fori_loop(0, tpw // 2, outer, 0)
    pltpu.sync_copy(res_v, out_hbm.at[pl.ds(wid * tpw * N, tpw * N)])


def _dedup_body(T, T1, N, blank, tc_hbm, sc_hbm, tok_hbm, len_hbm,
                row_v, sc_v, out_v, len_v):
    wid = lax.axis_index("s") * 2 + lax.axis_index("c")
    lane0 = lax.iota(jnp.int32, _LANES)
    pltpu.sync_copy(tc_hbm.at[wid], row_v.at[pl.ds(0, T1)])
    pltpu.sync_copy(sc_hbm, sc_v)

    base_idx = lane0 * N + wid

    def tail(jj, _):
        src = plsc.load_gather(sc_v, [jj * (_LANES * N) + base_idx])
        row_v[pl.ds(T1 + jj * _LANES, _LANES)] = src
        return 0

    lax.fori_loop(0, (T - T1) // _LANES, tail, 0)

    nchunks = T // _LANES
    lane = lax.iota(jnp.int32, _LANES)
    shift_idx = jnp.maximum(lane - 1, 0)
    last_idx = lane * 0 + (_LANES - 1)

    def fill(i, _):
        out_v[pl.ds(i * _LANES, _LANES)] = jnp.zeros((_LANES,), jnp.int32) - 1
        return 0

    lax.fori_loop(0, nchunks, fill, 0)

    def body(i, carry):
        total, prev_last = carry
        v = row_v[pl.ds(i * _LANES, _LANES)]
        shifted = _vgather(v, shift_idx)
        prev = jnp.where(lane == 0, prev_last, shifted)
        keep = (v != prev) & (v != blank)
        plsc.store_compressed(out_v.at[pl.ds(total, _LANES)], v, mask=keep)
        new_last = _vgather(v, last_idx)
        return total + jnp.sum(keep.astype(jnp.int32)), new_last

    init = (jnp.int32(0), jnp.zeros((_LANES,), jnp.int32) - 1)
    total, _ = lax.fori_loop(0, nchunks, body, init)

    pltpu.sync_copy(out_v, tok_hbm.at[wid])
    len_v[...] = jnp.zeros((_LANES,), jnp.int32) + total
    pltpu.sync_copy(len_v, len_hbm.at[wid])


def kernel(emission):
    T, N, C = emission.shape
    blank = C - 1
    TB = 128
    T1 = T - _T_SC
    sc_params = pltpu.CompilerParams(needs_layout_passes=False)
    mesh = plsc.VectorSubcoreMesh(core_axis_name="c", subcore_axis_name="s")

    sc_argmax = functools.partial(
        pl.kernel,
        mesh=mesh,
        out_type=jax.ShapeDtypeStruct((_T_SC * N,), jnp.int32),
        scratch_types=[
            pltpu.VMEM((N, C), jnp.float32),
            pltpu.VMEM((N, C), jnp.float32),
            pltpu.VMEM((_T_SC // 32 * N,), jnp.int32),
            pltpu.SemaphoreType.DMA,
            pltpu.SemaphoreType.DMA,
        ],
        compiler_params=sc_params,
    )(functools.partial(_sc_argmax_body, T1, N, C))
    idx_sc = sc_argmax(emission)

    idx_tc = pl.pallas_call(
        _argmax_body,
        grid=(T1 // TB,),
        in_specs=[pl.BlockSpec((TB, N, C), lambda i: (i, 0, 0))],
        out_specs=pl.BlockSpec((N, TB), lambda i: (0, i)),
        out_shape=jax.ShapeDtypeStruct((N, T1), jnp.int32),
    )(emission)

    dedup = functools.partial(
        pl.kernel,
        mesh=mesh,
        out_type=[
            jax.ShapeDtypeStruct((N, T), jnp.int32),
            jax.ShapeDtypeStruct((N, _LANES), jnp.int32),
        ],
        scratch_types=[
            pltpu.VMEM((T,), jnp.int32),
            pltpu.VMEM((_T_SC * N,), jnp.int32),
            pltpu.VMEM((T,), jnp.int32),
            pltpu.VMEM((_LANES,), jnp.int32),
        ],
        compiler_params=sc_params,
    )(functools.partial(_dedup_body, T, T1, N, blank))

    tokens, len_pad = dedup(idx_tc, idx_sc)
    lengths = len_pad[:, 0]
    return tokens, lengths


# speedup vs baseline: 1.0741x; 1.0741x over previous
import functools

import jax
import jax.numpy as jnp
from jax import lax
from jax.experimental import pallas as pl
from jax.experimental.pallas import tpu as pltpu
from jax.experimental.pallas import tpu_sc as plsc


_LANES = 16

_GATHER_DNUMS = lax.GatherDimensionNumbers(
    offset_dims=(), collapsed_slice_dims=(0,), start_index_map=(0,)
)


def _vgather(v, idx):
    return lax.gather(
        v,
        idx[:, None],
        _GATHER_DNUMS,
        slice_sizes=(1,),
        mode=lax.GatherScatterMode.PROMISE_IN_BOUNDS,
    )


def _argmax_body(x_ref, out_ref):
    x = x_ref[...]
    m = jnp.max(x, axis=-1, keepdims=True)
    ii = lax.broadcasted_iota(jnp.int32, x.shape, 2)
    C = x.shape[-1]
    idx = jnp.min(jnp.where(x >= m, ii, C), axis=-1).astype(jnp.int32)
    out_ref[...] = idx.T


def _dedup_body(T, blank, idx_hbm, tok_hbm, len_hbm, row_v, out_v, len_v):
    wid = lax.axis_index("s") * 2 + lax.axis_index("c")
    pltpu.sync_copy(idx_hbm.at[wid], row_v)

    nchunks = T // _LANES
    lane = lax.iota(jnp.int32, _LANES)
    shift_idx = jnp.maximum(lane - 1, 0)
    last_idx = lane * 0 + (_LANES - 1)

    def fill(i, _):
        out_v[pl.ds(i * _LANES, _LANES)] = jnp.zeros((_LANES,), jnp.int32) - 1
        return 0

    lax.fori_loop(0, nchunks, fill, 0)

    def body(i, carry):
        total, prev_last = carry
        v = row_v[pl.ds(i * _LANES, _LANES)]
        shifted = _vgather(v, shift_idx)
        prev = jnp.where(lane == 0, prev_last, shifted)
        keep = (v != prev) & (v != blank)
        plsc.store_compressed(out_v.at[pl.ds(total, _LANES)], v, mask=keep)
        new_last = _vgather(v, last_idx)
        return total + jnp.sum(keep.astype(jnp.int32)), new_last

    init = (jnp.int32(0), jnp.zeros((_LANES,), jnp.int32) - 1)
    total, _ = lax.fori_loop(0, nchunks, body, init)

    pltpu.sync_copy(out_v, tok_hbm.at[wid])
    len_v[...] = jnp.zeros((_LANES,), jnp.int32) + total
    pltpu.sync_copy(len_v, len_hbm.at[wid])


def kernel(emission):
    T, N, C = emission.shape
    blank = C - 1
    TB = 128

    idx = pl.pallas_call(
        _argmax_body,
        grid=(T // TB,),
        in_specs=[pl.BlockSpec((TB, N, C), lambda i: (i, 0, 0))],
        out_specs=pl.BlockSpec((N, TB), lambda i: (0, i)),
        out_shape=jax.ShapeDtypeStruct((N, T), jnp.int32),
    )(emission)

    mesh = plsc.VectorSubcoreMesh(core_axis_name="c", subcore_axis_name="s")
    dedup = functools.partial(
        pl.kernel,
        mesh=mesh,
        out_type=[
            jax.ShapeDtypeStruct((N, T), jnp.int32),
            jax.ShapeDtypeStruct((N, _LANES), jnp.int32),
        ],
        scratch_types=[
            pltpu.VMEM((T,), jnp.int32),
            pltpu.VMEM((T,), jnp.int32),
            pltpu.VMEM((_LANES,), jnp.int32),
        ],
        compiler_params=pltpu.CompilerParams(needs_layout_passes=False),
    )(functools.partial(_dedup_body, T, blank))

    tokens, len_pad = dedup(idx)
    lengths = len_pad[:, 0]
    return tokens, lengths
